# baseline (device time: 187952 ns/iter reference)
import jax
import jax.numpy as jnp
from jax import lax
from jax.experimental import pallas as pl
from jax.experimental.pallas import tpu as pltpu

N_DEV = 32
N_TOK = 1024
D_MODEL = 512
H = 1024
N_EXP = 128
EXP_PER_DEV = N_EXP // N_DEV
SEG = N_TOK // N_DEV


def kernel(x, router_W, route_idx, expert_W, shared_W):
    def body(
        x_ref,
        router_W_ref,
        route_idx_ref,
        expert_W_ref,
        shared_W_ref,
        out_ref,
        acc_ref,
        rs_scratch,
        rs_send_sems,
        rs_recv_sems,
        ag_send_sems,
        ag_recv_sems,
    ):
        me = lax.axis_index("i")
        left = lax.rem(me - 1 + N_DEV, N_DEV)
        right = lax.rem(me + 1, N_DEV)

        barrier_sem = pltpu.get_barrier_semaphore()
        for nbr in (left, right):
            pl.semaphore_signal(
                barrier_sem,
                inc=1,
                device_id=(nbr,),
                device_id_type=pl.DeviceIdType.MESH,
            )
        pl.semaphore_wait(barrier_sem, 2)

        x = x_ref[:, :]
        scores = jnp.dot(
            x,
            router_W_ref[:, :],
            preferred_element_type=jnp.float32,
            precision=lax.Precision.HIGHEST,
        )
        s_max = jnp.max(scores, axis=-1, keepdims=True)
        probs = jnp.exp(scores - s_max)
        probs = probs / jnp.sum(probs, axis=-1, keepdims=True)

        ridx = route_idx_ref[:, :]
        eids = lax.broadcasted_iota(jnp.int32, (1, N_EXP), 1)
        p = jnp.sum(
            jnp.where(ridx == eids, probs, 0.0), axis=-1, keepdims=True
        )

        partial = jnp.zeros((N_TOK, H), jnp.float32)
        for j in range(EXP_PER_DEV):
            sel = ridx == (me * EXP_PER_DEV + j)
            w = jnp.where(sel, p, 0.0)
            xw = (x * w).astype(jnp.bfloat16)
            partial = partial + jnp.dot(
                xw,
                expert_W_ref[j, :, :].astype(jnp.bfloat16),
                preferred_element_type=jnp.float32,
            )
        acc_ref[:, :] = partial.astype(jnp.bfloat16)

        for h in range(N_DEV - 1):
            send_seg = lax.rem(me - h + 2 * N_DEV, N_DEV)
            recv_seg = lax.rem(me - h - 1 + 2 * N_DEV, N_DEV)
            rdma = pltpu.make_async_remote_copy(
                src_ref=acc_ref.at[pl.ds(send_seg * SEG, SEG), :],
                dst_ref=rs_scratch.at[h],
                send_sem=rs_send_sems.at[h],
                recv_sem=rs_recv_sems.at[h],
                device_id=(right,),
                device_id_type=pl.DeviceIdType.MESH,
            )
            rdma.start()
            rdma.wait()
            acc_ref[pl.ds(recv_seg * SEG, SEG), :] = (
                acc_ref[pl.ds(recv_seg * SEG, SEG), :] + rs_scratch[h, :, :]
            )

        for h in range(N_DEV - 1):
            send_seg = lax.rem(me + 1 - h + 2 * N_DEV, N_DEV)
            rdma = pltpu.make_async_remote_copy(
                src_ref=acc_ref.at[pl.ds(send_seg * SEG, SEG), :],
                dst_ref=acc_ref.at[pl.ds(send_seg * SEG, SEG), :],
                send_sem=ag_send_sems.at[h],
                recv_sem=ag_recv_sems.at[h],
                device_id=(right,),
                device_id_type=pl.DeviceIdType.MESH,
            )
            rdma.start()
            rdma.wait()

        shared = jnp.dot(
            x.astype(jnp.bfloat16),
            shared_W_ref[:, :].astype(jnp.bfloat16),
            preferred_element_type=jnp.float32,
        )
        out_ref[:, :] = acc_ref[:, :].astype(jnp.float32) + shared

    return pl.pallas_call(
        body,
        out_shape=jax.ShapeDtypeStruct((N_TOK, H), jnp.float32),
        in_specs=[pl.BlockSpec(memory_space=pltpu.VMEM)] * 5,
        out_specs=pl.BlockSpec(memory_space=pltpu.VMEM),
        scratch_shapes=[
            pltpu.VMEM((N_TOK, H), jnp.bfloat16),
            pltpu.VMEM((N_DEV - 1, SEG, H), jnp.bfloat16),
            pltpu.SemaphoreType.DMA((N_DEV - 1,)),
            pltpu.SemaphoreType.DMA((N_DEV - 1,)),
            pltpu.SemaphoreType.DMA((N_DEV - 1,)),
            pltpu.SemaphoreType.DMA((N_DEV - 1,)),
        ],
        compiler_params=pltpu.CompilerParams(collective_id=0),
    )(x, router_W, route_idx, expert_W, shared_W)


# device time: 92690 ns/iter; 2.0277x vs baseline; 2.0277x over previous
import jax
import jax.numpy as jnp
from jax import lax
from jax.experimental import pallas as pl
from jax.experimental.pallas import tpu as pltpu

N_DEV = 32
N_TOK = 1024
D_MODEL = 512
H = 1024
N_EXP = 128
EXP_PER_DEV = N_EXP // N_DEV
SEG = N_TOK // N_DEV
N_STAGES = 5


def kernel(x, router_W, route_idx, expert_W, shared_W):
    def body(
        x_ref,
        router_W_ref,
        route_idx_ref,
        expert_W_ref,
        shared_W_ref,
        out_ref,
        acc_ref,
        rs_scratch,
        rs_send_sems,
        rs_recv_sems,
        ag_send_sems,
        ag_recv_sems,
    ):
        me = lax.axis_index("i")
        pz = me // 8
        pw = me % 8
        py = pw // 2
        px = (pw + py) % 2

        def to_logical(qx, qy, qz):
            return 8 * qz + 2 * qy + (qx ^ (qy % 2))

        partners = [
            to_logical(1 - px, py, pz),
            to_logical(px, py ^ 1, pz),
            to_logical(px, py, pz ^ 1),
            to_logical(px, py ^ 2, pz),
            to_logical(px, py, pz ^ 2),
        ]
        kls = [
            1 - px,
            1 - (py % 2),
            1 - (pz % 2),
            1 - (py // 2),
            1 - (pz // 2),
        ]
        half_segs = [16, 8, 4, 2, 1]
        scratch_off = [0, 512, 768, 896, 960]

        barrier_sem = pltpu.get_barrier_semaphore()
        for p in partners:
            pl.semaphore_signal(
                barrier_sem,
                inc=1,
                device_id=(p,),
                device_id_type=pl.DeviceIdType.MESH,
            )
        pl.semaphore_wait(barrier_sem, N_STAGES)

        x = x_ref[:, :]
        scores = jnp.dot(
            x,
            router_W_ref[:, :],
            preferred_element_type=jnp.float32,
            precision=lax.Precision.HIGHEST,
        )
        s_max = jnp.max(scores, axis=-1, keepdims=True)
        probs = jnp.exp(scores - s_max)
        probs = probs / jnp.sum(probs, axis=-1, keepdims=True)

        ridx = route_idx_ref[:, :]
        eids = lax.broadcasted_iota(jnp.int32, (1, N_EXP), 1)
        p_route = jnp.sum(
            jnp.where(ridx == eids, probs, 0.0), axis=-1, keepdims=True
        )

        partial = jnp.zeros((N_TOK, H), jnp.float32)
        for j in range(EXP_PER_DEV):
            sel = ridx == (me * EXP_PER_DEV + j)
            w = jnp.where(sel, p_route, 0.0)
            xw = (x * w).astype(jnp.bfloat16)
            partial = partial + jnp.dot(
                xw,
                expert_W_ref[j, :, :].astype(jnp.bfloat16),
                preferred_element_type=jnp.float32,
            )
        acc_ref[:, :] = partial.astype(jnp.bfloat16)

        base = me * 0
        for k in range(N_STAGES):
            half = half_segs[k]
            send_base = base + half * kls[k]
            rdma = pltpu.make_async_remote_copy(
                src_ref=acc_ref.at[pl.ds(send_base * SEG, half * SEG), :],
                dst_ref=rs_scratch.at[pl.ds(scratch_off[k], half * SEG), :],
                send_sem=rs_send_sems.at[k],
                recv_sem=rs_recv_sems.at[k],
                device_id=(partners[k],),
                device_id_type=pl.DeviceIdType.MESH,
            )
            rdma.start()
            rdma.wait()
            base = base + half * (1 - kls[k])
            acc_ref[pl.ds(base * SEG, half * SEG), :] = (
                acc_ref[pl.ds(base * SEG, half * SEG), :]
                + rs_scratch[pl.ds(scratch_off[k], half * SEG), :]
            )

        obase = base
        olen = 1
        for j in range(N_STAGES):
            k = N_STAGES - 1 - j
            rdma = pltpu.make_async_remote_copy(
                src_ref=acc_ref.at[pl.ds(obase * SEG, olen * SEG), :],
                dst_ref=acc_ref.at[pl.ds(obase * SEG, olen * SEG), :],
                send_sem=ag_send_sems.at[j],
                recv_sem=ag_recv_sems.at[j],
                device_id=(partners[k],),
                device_id_type=pl.DeviceIdType.MESH,
            )
            rdma.start()
            rdma.wait()
            obase = obase - olen * (1 - kls[k])
            olen *= 2

        shared = jnp.dot(
            x.astype(jnp.bfloat16),
            shared_W_ref[:, :].astype(jnp.bfloat16),
            preferred_element_type=jnp.float32,
        )
        out_ref[:, :] = acc_ref[:, :].astype(jnp.float32) + shared

    return pl.pallas_call(
        body,
        out_shape=jax.ShapeDtypeStruct((N_TOK, H), jnp.float32),
        in_specs=[pl.BlockSpec(memory_space=pltpu.VMEM)] * 5,
        out_specs=pl.BlockSpec(memory_space=pltpu.VMEM),
        scratch_shapes=[
            pltpu.VMEM((N_TOK, H), jnp.bfloat16),
            pltpu.VMEM((992, H), jnp.bfloat16),
            pltpu.SemaphoreType.DMA((N_STAGES,)),
            pltpu.SemaphoreType.DMA((N_STAGES,)),
            pltpu.SemaphoreType.DMA((N_STAGES,)),
            pltpu.SemaphoreType.DMA((N_STAGES,)),
        ],
        compiler_params=pltpu.CompilerParams(collective_id=0),
    )(x, router_W, route_idx, expert_W, shared_W)


# device time: 76263 ns/iter; 2.4645x vs baseline; 1.2154x over previous
import jax
import jax.numpy as jnp
from jax import lax
from jax.experimental import pallas as pl
from jax.experimental.pallas import tpu as pltpu

N_DEV = 32
N_TOK = 1024
D_MODEL = 512
H = 1024
N_EXP = 128
EXP_PER_DEV = N_EXP // N_DEV
N_STAGES = 5
BLK = 16

GROUPS = [
    (0, (0, 1, 2, 3, 4), (0, 256, 384, 448, 480)),
    (512, (1, 0, 3, 4, 2), (512, 768, 896, 960, 992)),
]


def kernel(x, router_W, route_idx, expert_W, shared_W):
    def body(
        x_ref,
        router_W_ref,
        route_idx_ref,
        expert_W_ref,
        shared_W_ref,
        out_ref,
        acc_ref,
        rs_scratch,
        rs_send_sems,
        rs_recv_sems,
        ag_send_sems,
        ag_recv_sems,
    ):
        me = lax.axis_index("i")
        pz = me // 8
        pw = me % 8
        py = pw // 2
        px = (pw + py) % 2

        def to_logical(qx, qy, qz):
            return 8 * qz + 2 * qy + (qx ^ (qy % 2))

        partners = [
            to_logical(1 - px, py, pz),
            to_logical(px, py ^ 1, pz),
            to_logical(px, py, pz ^ 1),
            to_logical(px, py ^ 2, pz),
            to_logical(px, py, pz ^ 2),
        ]
        kls = [
            1 - px,
            1 - (py % 2),
            1 - (pz % 2),
            1 - (py // 2),
            1 - (pz // 2),
        ]

        barrier_sem = pltpu.get_barrier_semaphore()
        for p in partners:
            pl.semaphore_signal(
                barrier_sem,
                inc=1,
                device_id=(p,),
                device_id_type=pl.DeviceIdType.MESH,
            )
        pl.semaphore_wait(barrier_sem, N_STAGES)

        x = x_ref[:, :]
        scores = jnp.dot(
            x,
            router_W_ref[:, :],
            preferred_element_type=jnp.float32,
            precision=lax.Precision.HIGHEST,
        )
        s_max = jnp.max(scores, axis=-1, keepdims=True)
        probs = jnp.exp(scores - s_max)
        probs = probs / jnp.sum(probs, axis=-1, keepdims=True)

        ridx = route_idx_ref[:, :]
        eids = lax.broadcasted_iota(jnp.int32, (1, N_EXP), 1)
        p_route = jnp.sum(
            jnp.where(ridx == eids, probs, 0.0), axis=-1, keepdims=True
        )

        partial = jnp.zeros((N_TOK, H), jnp.float32)
        for j in range(EXP_PER_DEV):
            sel = ridx == (me * EXP_PER_DEV + j)
            w = jnp.where(sel, p_route, 0.0)
            xw = (x * w).astype(jnp.bfloat16)
            partial = partial + jnp.dot(
                xw,
                expert_W_ref[j, :, :].astype(jnp.bfloat16),
                preferred_element_type=jnp.float32,
            )
        acc_ref[:, :] = partial.astype(jnp.bfloat16)

        bases = [me * 0, me * 0]
        for s in range(N_STAGES):
            half = 16 >> s
            rdmas = []
            for g, (row0, order, scr) in enumerate(GROUPS):
                pr = order[s]
                send_base = bases[g] + half * kls[pr]
                rdma = pltpu.make_async_remote_copy(
                    src_ref=acc_ref.at[
                        pl.ds(row0 + send_base * BLK, half * BLK), :
                    ],
                    dst_ref=rs_scratch.at[pl.ds(scr[s], half * BLK), :],
                    send_sem=rs_send_sems.at[g, s],
                    recv_sem=rs_recv_sems.at[g, s],
                    device_id=(partners[pr],),
                    device_id_type=pl.DeviceIdType.MESH,
                )
                rdma.start()
                rdmas.append(rdma)
            for g, (row0, order, scr) in enumerate(GROUPS):
                rdmas[g].wait()
                pr = order[s]
                bases[g] = bases[g] + half * (1 - kls[pr])
                kept = pl.ds(row0 + bases[g] * BLK, half * BLK)
                acc_ref[kept, :] = (
                    acc_ref[kept, :]
                    + rs_scratch[pl.ds(scr[s], half * BLK), :]
                )

        obases = list(bases)
        for j in range(N_STAGES):
            olen = 1 << j
            rdmas = []
            for g, (row0, order, _) in enumerate(GROUPS):
                pr = order[N_STAGES - 1 - j]
                rdma = pltpu.make_async_remote_copy(
                    src_ref=acc_ref.at[
                        pl.ds(row0 + obases[g] * BLK, olen * BLK), :
                    ],
                    dst_ref=acc_ref.at[
                        pl.ds(row0 + obases[g] * BLK, olen * BLK), :
                    ],
                    send_sem=ag_send_sems.at[g, j],
                    recv_sem=ag_recv_sems.at[g, j],
                    device_id=(partners[pr],),
                    device_id_type=pl.DeviceIdType.MESH,
                )
                rdma.start()
                rdmas.append(rdma)
            for g, (row0, order, _) in enumerate(GROUPS):
                rdmas[g].wait()
                pr = order[N_STAGES - 1 - j]
                obases[g] = obases[g] - olen * (1 - kls[pr])

        shared = jnp.dot(
            x.astype(jnp.bfloat16),
            shared_W_ref[:, :].astype(jnp.bfloat16),
            preferred_element_type=jnp.float32,
        )
        out_ref[:, :] = acc_ref[:, :].astype(jnp.float32) + shared

    return pl.pallas_call(
        body,
        out_shape=jax.ShapeDtypeStruct((N_TOK, H), jnp.float32),
        in_specs=[pl.BlockSpec(memory_space=pltpu.VMEM)] * 5,
        out_specs=pl.BlockSpec(memory_space=pltpu.VMEM),
        scratch_shapes=[
            pltpu.VMEM((N_TOK, H), jnp.bfloat16),
            pltpu.VMEM((1024, H), jnp.bfloat16),
            pltpu.SemaphoreType.DMA((2, N_STAGES)),
            pltpu.SemaphoreType.DMA((2, N_STAGES)),
            pltpu.SemaphoreType.DMA((2, N_STAGES)),
            pltpu.SemaphoreType.DMA((2, N_STAGES)),
        ],
        compiler_params=pltpu.CompilerParams(collective_id=0),
    )(x, router_W, route_idx, expert_W, shared_W)
